# baseline (speedup 1.0000x reference)
import math

import jax
import jax.numpy as jnp
from jax import lax
from jax.experimental import pallas as pl
from jax.experimental.pallas import tpu as pltpu
from jax.experimental.pallas import tpu_sc as plsc

B, L, V, H, E = 2, 512, 16, 8, 32
D = H * E
TOP_K = int(math.log(L))
NEG = -1e30


def _roll_up_lanes(x, s):
    return jnp.concatenate([x[:, s:], x[:, :s]], axis=1)


def _roll_right_lanes(x, s):
    n = x.shape[1]
    return jnp.concatenate([x[:, n - s:], x[:, :n - s]], axis=1)


def _corr_kernel(q_ref, k_ref, c_ref):
    for v in range(V):
        q = q_ref[0, :, v, :]
        k = k_ref[0, :, v, :]
        m = jax.lax.dot_general(
            k, q, (((1,), (1,)), ((), ())),
            preferred_element_type=jnp.float32,
            precision=jax.lax.Precision.HIGHEST,
        )
        h = L // 2
        while h >= 1:
            m = m[:h, :] + _roll_up_lanes(m[h:, :], h)
            h //= 2
        c_ref[0, v, :] = m[0, :] * (1.0 / D)


_NCH = L // 16


_GDN = lax.GatherDimensionNumbers(
    offset_dims=(), collapsed_slice_dims=(0,), start_index_map=(0,))


def _sc_select_body(c_hbm, p_hbm, cv0, cv1, wref, pref):
    wid = lax.axis_index("s") * 2 + lax.axis_index("c")
    lanes = lax.iota(jnp.int32, 16)

    def shuf(vec, s):
        idx = jnp.bitwise_xor(lanes, s)[:, None]
        return lax.gather(vec, idx, _GDN, (1,),
                          mode=lax.GatherScatterMode.PROMISE_IN_BOUNDS)

    def bcast_reduce(vec, op):
        m = vec
        for s in (1, 2, 4, 8):
            m = op(m, shuf(m, s))
        return m

    @pl.when(wid < V)
    def _():
        v = wid
        pltpu.sync_copy(c_hbm.at[0, v], cv0)
        pltpu.sync_copy(c_hbm.at[1, v], cv1)
        for i in range(_NCH):
            sl = pl.ds(i * 16, 16)
            wref[sl] = (cv0[sl] + cv1[sl]) * 0.5
        lanesf = lanes.astype(jnp.float32)
        negs = jnp.full((16,), NEG, jnp.float32)
        zeros = jnp.zeros((16,), jnp.float32)
        bigidx = jnp.full((16,), float(L), jnp.float32)
        amvs = []
        wv0 = negs
        wv1 = negs
        for k in range(TOP_K):
            m = negs
            for i in range(_NCH):
                m = jnp.maximum(m, wref[pl.ds(i * 16, 16)])
            rmv = bcast_reduce(m, jnp.maximum)
            amv = bigidx
            for i in range(_NCH):
                chunk = wref[pl.ds(i * 16, 16)]
                amv = jnp.minimum(
                    amv, jnp.where(chunk == rmv, lanesf + (i * 16), bigidx))
            amv = bcast_reduce(amv, jnp.minimum)
            a0 = zeros
            a1 = zeros
            for i in range(_NCH):
                sl = pl.ds(i * 16, 16)
                hit = (lanesf + (i * 16)) == amv
                wref[sl] = jnp.where(hit, negs, wref[sl])
                a0 = a0 + jnp.where(hit, cv0[sl], zeros)
                a1 = a1 + jnp.where(hit, cv1[sl], zeros)
            isk = lanes == k
            wv0 = jnp.where(isk, bcast_reduce(a0, jnp.add), wv0)
            wv1 = jnp.where(isk, bcast_reduce(a1, jnp.add), wv1)
            amvs.append(amv)
        kmask = lanes < TOP_K
        for b, wv in ((0, wv0), (1, wv1)):
            mxv = bcast_reduce(wv, jnp.maximum)
            ex = jnp.where(kmask, jnp.exp(wv - mxv), zeros)
            sv = bcast_reduce(ex, jnp.add)
            p16 = ex / sv
            pks = [bcast_reduce(jnp.where(lanes == k, p16, zeros), jnp.add)
                   for k in range(TOP_K)]
            for i in range(_NCH):
                row = zeros
                for k in range(TOP_K):
                    row = jnp.where((lanesf + (i * 16)) == amvs[k], pks[k], row)
                pref[pl.ds(i * 16, 16)] = row
            pltpu.sync_copy(pref, p_hbm.at[b, v])


def _sc_select(c):
    return pl.kernel(
        _sc_select_body,
        out_type=jax.ShapeDtypeStruct((B, V, L), jnp.float32),
        mesh=plsc.VectorSubcoreMesh(core_axis_name="c", subcore_axis_name="s"),
        scratch_types=[
            pltpu.VMEM((L,), jnp.float32),
            pltpu.VMEM((L,), jnp.float32),
            pltpu.VMEM((L,), jnp.float32),
            pltpu.VMEM((L,), jnp.float32),
        ],
    )(c)


def _agg_kernel(p_ref, v_ref, o_ref):
    prow = jax.lax.broadcasted_iota(jnp.int32, (D, D), 0)
    pcol = jax.lax.broadcasted_iota(jnp.int32, (D, D), 1)
    perm = (pcol == (prow % E) * H + prow // E).astype(jnp.float32)
    for v in range(V):
        p = p_ref[0, v:v + 1, :]
        cmat = p
        s = 1
        while s < L:
            cmat = jnp.concatenate([cmat, _roll_right_lanes(cmat, s)], axis=0)
            s *= 2
        vals = v_ref[0, :, v, :]
        agg = jax.lax.dot_general(
            cmat, vals, (((1,), (0,)), ((), ())),
            preferred_element_type=jnp.float32,
            precision=jax.lax.Precision.DEFAULT,
        )
        o_ref[0, :, v, :] = jax.lax.dot_general(
            agg, perm, (((1,), (0,)), ((), ())),
            preferred_element_type=jnp.float32,
            precision=jax.lax.Precision.DEFAULT,
        )


@jax.jit
def kernel(queries, keys, values):
    qr = queries.reshape(B, L, V, D)
    kr = keys.reshape(B, L, V, D)
    vr = values.reshape(B, L, V, D)

    c = pl.pallas_call(
        _corr_kernel,
        grid=(B,),
        in_specs=[
            pl.BlockSpec((1, L, V, D), lambda b: (b, 0, 0, 0)),
            pl.BlockSpec((1, L, V, D), lambda b: (b, 0, 0, 0)),
        ],
        out_specs=pl.BlockSpec((1, V, L), lambda b: (b, 0, 0)),
        out_shape=jax.ShapeDtypeStruct((B, V, L), jnp.float32),
    )(qr, kr)

    p = _sc_select(c)

    out = pl.pallas_call(
        _agg_kernel,
        grid=(B,),
        in_specs=[
            pl.BlockSpec((1, V, L), lambda b: (b, 0, 0)),
            pl.BlockSpec((1, L, V, D), lambda b: (b, 0, 0, 0)),
        ],
        out_specs=pl.BlockSpec((1, L, V, D), lambda b: (b, 0, 0, 0)),
        out_shape=jax.ShapeDtypeStruct((B, L, V, D), jnp.float32),
    )(p, vr)

    return out.reshape(B, L, V, E, H)


# speedup vs baseline: 1.0606x; 1.0606x over previous
import math

import jax
import jax.numpy as jnp
from jax import lax
from jax.experimental import pallas as pl
from jax.experimental.pallas import tpu as pltpu
from jax.experimental.pallas import tpu_sc as plsc

B, L, V, H, E = 2, 512, 16, 8, 32
D = H * E
TOP_K = int(math.log(L))
NEG = -1e30


def _roll_up_lanes(x, s):
    return jnp.concatenate([x[:, s:], x[:, :s]], axis=1)


def _roll_right_lanes(x, s):
    n = x.shape[1]
    return jnp.concatenate([x[:, n - s:], x[:, :n - s]], axis=1)


def _corr_kernel(q_ref, k_ref, c_ref):
    for v in range(V // 2):
        q = q_ref[0, :, v, :]
        k = k_ref[0, :, v, :]
        m = jax.lax.dot_general(
            k, q, (((1,), (1,)), ((), ())),
            preferred_element_type=jnp.float32,
            precision=jax.lax.Precision.HIGHEST,
        )
        h = L // 2
        while h >= 1:
            m = m[:h, :] + _roll_up_lanes(m[h:, :], h)
            h //= 2
        c_ref[0, v, :] = m[0, :] * (1.0 / D)


_NCH = L // 16


_GDN = lax.GatherDimensionNumbers(
    offset_dims=(), collapsed_slice_dims=(0,), start_index_map=(0,))


def _sc_select_body(c_hbm, p_hbm, cv0, cv1, wref, pref):
    wid = lax.axis_index("s") * 2 + lax.axis_index("c")
    lanes = lax.iota(jnp.int32, 16)

    def shuf(vec, s):
        idx = jnp.bitwise_xor(lanes, s)[:, None]
        return lax.gather(vec, idx, _GDN, (1,),
                          mode=lax.GatherScatterMode.PROMISE_IN_BOUNDS)

    def bcast_reduce(vec, op):
        m = vec
        for s in (1, 2, 4, 8):
            m = op(m, shuf(m, s))
        return m

    @pl.when(wid < V)
    def _():
        v = wid
        pltpu.sync_copy(c_hbm.at[0, v], cv0)
        pltpu.sync_copy(c_hbm.at[1, v], cv1)
        for i in range(_NCH):
            sl = pl.ds(i * 16, 16)
            wref[sl] = (cv0[sl] + cv1[sl]) * 0.5
        lanesf = lanes.astype(jnp.float32)
        negs = jnp.full((16,), NEG, jnp.float32)
        zeros = jnp.zeros((16,), jnp.float32)
        bigidx = jnp.full((16,), float(L), jnp.float32)
        amvs = []
        wv0 = negs
        wv1 = negs
        for k in range(TOP_K):
            m = negs
            for i in range(_NCH):
                m = jnp.maximum(m, wref[pl.ds(i * 16, 16)])
            rmv = bcast_reduce(m, jnp.maximum)
            amv = bigidx
            for i in range(_NCH):
                chunk = wref[pl.ds(i * 16, 16)]
                amv = jnp.minimum(
                    amv, jnp.where(chunk == rmv, lanesf + (i * 16), bigidx))
            amv = bcast_reduce(amv, jnp.minimum)
            a0 = zeros
            a1 = zeros
            for i in range(_NCH):
                sl = pl.ds(i * 16, 16)
                hit = (lanesf + (i * 16)) == amv
                wref[sl] = jnp.where(hit, negs, wref[sl])
                a0 = a0 + jnp.where(hit, cv0[sl], zeros)
                a1 = a1 + jnp.where(hit, cv1[sl], zeros)
            isk = lanes == k
            wv0 = jnp.where(isk, bcast_reduce(a0, jnp.add), wv0)
            wv1 = jnp.where(isk, bcast_reduce(a1, jnp.add), wv1)
            amvs.append(amv)
        kmask = lanes < TOP_K
        for b, wv in ((0, wv0), (1, wv1)):
            mxv = bcast_reduce(wv, jnp.maximum)
            ex = jnp.where(kmask, jnp.exp(wv - mxv), zeros)
            sv = bcast_reduce(ex, jnp.add)
            p16 = ex / sv
            pks = [bcast_reduce(jnp.where(lanes == k, p16, zeros), jnp.add)
                   for k in range(TOP_K)]
            for i in range(_NCH):
                row = zeros
                for k in range(TOP_K):
                    row = jnp.where((lanesf + (i * 16)) == amvs[k], pks[k], row)
                pref[pl.ds(i * 16, 16)] = row
            pltpu.sync_copy(pref, p_hbm.at[b, v])


def _sc_select(c):
    return pl.kernel(
        _sc_select_body,
        out_type=jax.ShapeDtypeStruct((B, V, L), jnp.float32),
        mesh=plsc.VectorSubcoreMesh(core_axis_name="c", subcore_axis_name="s"),
        scratch_types=[
            pltpu.VMEM((L,), jnp.float32),
            pltpu.VMEM((L,), jnp.float32),
            pltpu.VMEM((L,), jnp.float32),
            pltpu.VMEM((L,), jnp.float32),
        ],
    )(c)


def _agg_kernel(p_ref, v_ref, o_ref):
    prow = jax.lax.broadcasted_iota(jnp.int32, (D, D), 0)
    pcol = jax.lax.broadcasted_iota(jnp.int32, (D, D), 1)
    perm = (pcol == (prow % E) * H + prow // E).astype(jnp.float32)
    for v in range(V // 2):
        p = p_ref[0, v:v + 1, :]
        cmat = p
        s = 1
        while s < L:
            cmat = jnp.concatenate([cmat, _roll_right_lanes(cmat, s)], axis=0)
            s *= 2
        vals = v_ref[0, :, v, :]
        agg = jax.lax.dot_general(
            cmat, vals, (((1,), (0,)), ((), ())),
            preferred_element_type=jnp.float32,
            precision=jax.lax.Precision.DEFAULT,
        )
        o_ref[0, :, v, :] = jax.lax.dot_general(
            agg, perm, (((1,), (0,)), ((), ())),
            preferred_element_type=jnp.float32,
            precision=jax.lax.Precision.DEFAULT,
        )


@jax.jit
def kernel(queries, keys, values):
    qr = queries.reshape(B, L, V, D)
    kr = keys.reshape(B, L, V, D)
    vr = values.reshape(B, L, V, D)

    c = pl.pallas_call(
        _corr_kernel,
        grid=(B, 2),
        in_specs=[
            pl.BlockSpec((1, L, V // 2, D), lambda b, i: (b, 0, i, 0)),
            pl.BlockSpec((1, L, V // 2, D), lambda b, i: (b, 0, i, 0)),
        ],
        out_specs=pl.BlockSpec((1, V // 2, L), lambda b, i: (b, i, 0)),
        out_shape=jax.ShapeDtypeStruct((B, V, L), jnp.float32),
    )(qr, kr)

    p = _sc_select(c)

    out = pl.pallas_call(
        _agg_kernel,
        grid=(B, 2),
        in_specs=[
            pl.BlockSpec((1, V // 2, L), lambda b, i: (b, i, 0)),
            pl.BlockSpec((1, L, V // 2, D), lambda b, i: (b, 0, i, 0)),
        ],
        out_specs=pl.BlockSpec((1, L, V // 2, D), lambda b, i: (b, 0, i, 0)),
        out_shape=jax.ShapeDtypeStruct((B, L, V, D), jnp.float32),
    )(p, vr)

    return out.reshape(B, L, V, E, H)
